# Initial kernel scaffold; baseline (speedup 1.0000x reference)
import jax
import jax.numpy as jnp
from jax.experimental import pallas as pl


def kernel(h, E, e_dists, null_context, e_t, e_idx, n_entities, e_len, W_R, W_Ectx, lam, W_L, b_L, entity_init_mean, W_forget, W_input, W_X, W_Xnull):
    raise NotImplementedError("write your pallas kernel here")


# speedup vs baseline: 7.7619x; 7.7619x over previous
import jax
import jax.numpy as jnp
from jax.experimental import pallas as pl

B = 4096
HID = 256
ED = 256
MAX_E = 64
MAX_LEN = 25
R = 128


def _body(h_ref, E_ref, ed_ref, null_ref, et_ref, eidx_ref, nent_ref,
          noise_ref, mean_ref, lam_ref, bL_ref,
          WR_ref, WEctx_ref, WL1_ref, WL2_ref, Wf_ref, Wi_ref, WX_ref, WXn_ref,
          out_t_ref, out_idx_ref, out_len_ref, out_x_ref):
    h = h_ref[:]
    E3 = E_ref[:]
    eidx = eidx_ref[:]
    nent = nent_ref[:]
    et = et_ref[:]

    z = mean_ref[:] + noise_ref[:] * 0.0001
    z = z / jnp.sqrt(jnp.sum(z * z, axis=1, keepdims=True))

    proj_e = jnp.dot(h, WEctx_ref[:], preferred_element_type=jnp.float32)
    proj_f = jnp.dot(h, Wf_ref[:], preferred_element_type=jnp.float32)
    ivec = jnp.dot(h, Wi_ref[:], preferred_element_type=jnp.float32)

    base = jnp.sum(E3 * proj_e[:, None, :], axis=2)
    iota = jax.lax.broadcasted_iota(jnp.int32, (R, MAX_E), 1)
    oh = (iota == eidx).astype(jnp.float32)
    gathered0 = jnp.sum(E3 * oh[:, :, None], axis=1)

    add_mask = jnp.logical_and(eidx >= nent, nent < MAX_E)
    col_add = jnp.clip(nent, 0, MAX_E - 1)
    add_hit = jnp.logical_and(add_mask, eidx == col_add)
    e_mask = (et == 1)

    gathered = jnp.where(add_hit, z, gathered0)
    f = jnp.sum(gathered * proj_f, axis=1, keepdims=True)
    upd = (1.0 - f) * gathered + f * ivec
    curr_e = jnp.where(e_mask, upd, gathered)

    dot_z = jnp.sum(z * proj_e, axis=1, keepdims=True)
    dot_upd = jnp.sum(upd * proj_e, axis=1, keepdims=True)
    oidx = base
    oidx = jnp.where(jnp.logical_and(add_mask, iota == col_add), dot_z, oidx)
    oidx = jnp.where(jnp.logical_and(e_mask, iota == eidx), dot_upd, oidx)
    oidx = oidx + jnp.exp(ed_ref[:] * lam_ref[0, 0])
    out_idx_ref[:] = oidx

    out_t_ref[:] = jnp.dot(h, WR_ref[:], preferred_element_type=jnp.float32)
    out_len_ref[:] = (jnp.dot(h, WL1_ref[:], preferred_element_type=jnp.float32)
                      + jnp.dot(curr_e, WL2_ref[:], preferred_element_type=jnp.float32)
                      + bL_ref[:])
    xa = jnp.dot(curr_e, WX_ref[:], preferred_element_type=jnp.float32)
    xb = jnp.dot(null_ref[:], WXn_ref[:], preferred_element_type=jnp.float32)
    out_x_ref[:] = jnp.where(e_mask, xa, xb)


def kernel(h, E, e_dists, null_context, e_t, e_idx, n_entities, e_len,
           W_R, W_Ectx, lam, W_L, b_L, entity_init_mean,
           W_forget, W_input, W_X, W_Xnull):
    del e_len
    noise = jax.random.normal(jax.random.key(42), (B, ED), jnp.float32)
    grid = (B // R,)
    full = lambda shape: pl.BlockSpec(shape, lambda b: (0,) * len(shape))

    out_shapes = (
        jax.ShapeDtypeStruct((B, 2), jnp.float32),
        jax.ShapeDtypeStruct((B, MAX_E), jnp.float32),
        jax.ShapeDtypeStruct((B, MAX_LEN), jnp.float32),
        jax.ShapeDtypeStruct((B, ED), jnp.float32),
    )
    in_specs = [
        pl.BlockSpec((R, HID), lambda b: (b, 0)),
        pl.BlockSpec((R, MAX_E, ED), lambda b: (b, 0, 0)),
        pl.BlockSpec((R, MAX_E), lambda b: (b, 0)),
        pl.BlockSpec((R, ED), lambda b: (b, 0)),
        pl.BlockSpec((R, 1), lambda b: (b, 0)),
        pl.BlockSpec((R, 1), lambda b: (b, 0)),
        pl.BlockSpec((R, 1), lambda b: (b, 0)),
        pl.BlockSpec((R, ED), lambda b: (b, 0)),
        full((1, ED)),
        full((1, 1)),
        full((1, MAX_LEN)),
        full((HID, 2)),
        full((HID, ED)),
        full((HID, MAX_LEN)),
        full((ED, MAX_LEN)),
        full((HID, ED)),
        full((HID, ED)),
        full((ED, HID)),
        full((ED, HID)),
    ]
    out_specs = (
        pl.BlockSpec((R, 2), lambda b: (b, 0)),
        pl.BlockSpec((R, MAX_E), lambda b: (b, 0)),
        pl.BlockSpec((R, MAX_LEN), lambda b: (b, 0)),
        pl.BlockSpec((R, ED), lambda b: (b, 0)),
    )

    return pl.pallas_call(
        _body,
        grid=grid,
        in_specs=in_specs,
        out_specs=out_specs,
        out_shape=out_shapes,
    )(
        h, E, e_dists, null_context,
        e_t.reshape(B, 1), e_idx.reshape(B, 1), n_entities.reshape(B, 1),
        noise, entity_init_mean.reshape(1, ED), lam.reshape(1, 1),
        b_L.reshape(1, MAX_LEN),
        W_R.T, W_Ectx.T, W_L[:, :HID].T, W_L[:, HID:].T,
        W_forget.T, W_input.T, W_X.T, W_Xnull.T,
    )


# speedup vs baseline: 8.1095x; 1.0448x over previous
import jax
import jax.numpy as jnp
from jax.experimental import pallas as pl

B = 4096
HID = 256
ED = 256
MAX_E = 64
MAX_LEN = 25
R = 256


def _body(h_ref, E_ref, ed_ref, null_ref, et_ref, eidx_ref, nent_ref,
          noise_ref, mean_ref, lam_ref, bL_ref,
          WR_ref, WEctx_ref, WL1_ref, WL2_ref, Wf_ref, Wi_ref, WX_ref, WXn_ref,
          out_t_ref, out_idx_ref, out_len_ref, out_x_ref):
    h = h_ref[:]
    E3 = E_ref[:]
    eidx = eidx_ref[:]
    nent = nent_ref[:]
    et = et_ref[:]

    z = mean_ref[:] + noise_ref[:] * 0.0001
    z = z / jnp.sqrt(jnp.sum(z * z, axis=1, keepdims=True))

    proj_e = jnp.dot(h, WEctx_ref[:], preferred_element_type=jnp.float32)
    proj_f = jnp.dot(h, Wf_ref[:], preferred_element_type=jnp.float32)
    ivec = jnp.dot(h, Wi_ref[:], preferred_element_type=jnp.float32)

    base = jnp.sum(E3 * proj_e[:, None, :], axis=2)
    iota = jax.lax.broadcasted_iota(jnp.int32, (R, MAX_E), 1)
    oh = (iota == eidx).astype(jnp.float32)
    gathered0 = jnp.sum(E3 * oh[:, :, None], axis=1)

    add_mask = jnp.logical_and(eidx >= nent, nent < MAX_E)
    col_add = jnp.clip(nent, 0, MAX_E - 1)
    add_hit = jnp.logical_and(add_mask, eidx == col_add)
    e_mask = (et == 1)

    gathered = jnp.where(add_hit, z, gathered0)
    f = jnp.sum(gathered * proj_f, axis=1, keepdims=True)
    upd = (1.0 - f) * gathered + f * ivec
    curr_e = jnp.where(e_mask, upd, gathered)

    dot_z = jnp.sum(z * proj_e, axis=1, keepdims=True)
    dot_upd = jnp.sum(upd * proj_e, axis=1, keepdims=True)
    oidx = base
    oidx = jnp.where(jnp.logical_and(add_mask, iota == col_add), dot_z, oidx)
    oidx = jnp.where(jnp.logical_and(e_mask, iota == eidx), dot_upd, oidx)
    oidx = oidx + jnp.exp(ed_ref[:] * lam_ref[0, 0])
    out_idx_ref[:] = oidx

    out_t_ref[:] = jnp.dot(h, WR_ref[:], preferred_element_type=jnp.float32)
    out_len_ref[:] = (jnp.dot(h, WL1_ref[:], preferred_element_type=jnp.float32)
                      + jnp.dot(curr_e, WL2_ref[:], preferred_element_type=jnp.float32)
                      + bL_ref[:])
    xa = jnp.dot(curr_e, WX_ref[:], preferred_element_type=jnp.float32)
    xb = jnp.dot(null_ref[:], WXn_ref[:], preferred_element_type=jnp.float32)
    out_x_ref[:] = jnp.where(e_mask, xa, xb)


def kernel(h, E, e_dists, null_context, e_t, e_idx, n_entities, e_len,
           W_R, W_Ectx, lam, W_L, b_L, entity_init_mean,
           W_forget, W_input, W_X, W_Xnull):
    del e_len
    noise = jax.random.normal(jax.random.key(42), (B, ED), jnp.float32)
    grid = (B // R,)
    full = lambda shape: pl.BlockSpec(shape, lambda b: (0,) * len(shape))

    out_shapes = (
        jax.ShapeDtypeStruct((B, 2), jnp.float32),
        jax.ShapeDtypeStruct((B, MAX_E), jnp.float32),
        jax.ShapeDtypeStruct((B, MAX_LEN), jnp.float32),
        jax.ShapeDtypeStruct((B, ED), jnp.float32),
    )
    in_specs = [
        pl.BlockSpec((R, HID), lambda b: (b, 0)),
        pl.BlockSpec((R, MAX_E, ED), lambda b: (b, 0, 0)),
        pl.BlockSpec((R, MAX_E), lambda b: (b, 0)),
        pl.BlockSpec((R, ED), lambda b: (b, 0)),
        pl.BlockSpec((R, 1), lambda b: (b, 0)),
        pl.BlockSpec((R, 1), lambda b: (b, 0)),
        pl.BlockSpec((R, 1), lambda b: (b, 0)),
        pl.BlockSpec((R, ED), lambda b: (b, 0)),
        full((1, ED)),
        full((1, 1)),
        full((1, MAX_LEN)),
        full((HID, 2)),
        full((HID, ED)),
        full((HID, MAX_LEN)),
        full((ED, MAX_LEN)),
        full((HID, ED)),
        full((HID, ED)),
        full((ED, HID)),
        full((ED, HID)),
    ]
    out_specs = (
        pl.BlockSpec((R, 2), lambda b: (b, 0)),
        pl.BlockSpec((R, MAX_E), lambda b: (b, 0)),
        pl.BlockSpec((R, MAX_LEN), lambda b: (b, 0)),
        pl.BlockSpec((R, ED), lambda b: (b, 0)),
    )

    return pl.pallas_call(
        _body,
        grid=grid,
        in_specs=in_specs,
        out_specs=out_specs,
        out_shape=out_shapes,
    )(
        h, E, e_dists, null_context,
        e_t.reshape(B, 1), e_idx.reshape(B, 1), n_entities.reshape(B, 1),
        noise, entity_init_mean.reshape(1, ED), lam.reshape(1, 1),
        b_L.reshape(1, MAX_LEN),
        W_R.T, W_Ectx.T, W_L[:, :HID].T, W_L[:, HID:].T,
        W_forget.T, W_input.T, W_X.T, W_Xnull.T,
    )
